# baseline (speedup 1.0000x reference)
import functools

import jax
import jax.numpy as jnp
from jax import lax
from jax.experimental import pallas as pl
from jax.experimental.pallas import tpu as pltpu
from jax.experimental.pallas import tpu_sc as plsc

_N = 20000
_G = 100
_C = 80
_P = 512
_LANES = 16
_TILES = 16
_CHUNK = 1264
_NPAD = _TILES * _CHUNK
_VPT = _P // (_TILES * _LANES)
_FIFTH_ROOT_MAGIC = 851980270


def _sc_body(pos_hbm, pblk_hbm, cls_hbm, gtb_hbm, gtl_hbm, out_hbm,
             zbuf, idx_v, blkidx_v, blk_v, crows_v, gtb_v, gtl_v, qbuf, sem):
    c = lax.axis_index("c")
    s = lax.axis_index("s")

    @pl.when(c == 0)
    def _work():
        def _zfill(i, carry):
            zbuf[pl.ds(i * _LANES, _LANES)] = jnp.zeros((_LANES,), jnp.float32)
            return carry

        lax.fori_loop(0, _CHUNK // _LANES, _zfill, 0)
        pltpu.sync_copy(zbuf, out_hbm.at[pl.ds(s * _CHUNK, _CHUNK)])
        plsc.subcore_barrier()

        pltpu.sync_copy(gtb_hbm, gtb_v)
        pltpu.sync_copy(gtl_hbm, gtl_v)

        lane = lax.iota(jnp.int32, _LANES)
        col0 = jnp.zeros((_LANES,), jnp.int32)

        for v in range(_VPT):
            base = (s * _VPT + v) * _LANES
            pltpu.sync_copy(pos_hbm.at[pl.ds(base, _LANES)], idx_v)
            idx = idx_v[...]
            blkidx_v[...] = idx // 4
            pltpu.async_copy(pblk_hbm.at[blkidx_v], blk_v, sem).wait()
            pltpu.async_copy(cls_hbm.at[idx_v], crows_v, sem).wait()

            sub = (idx % 4) * 4
            px1 = plsc.load_gather(blk_v, [lane, sub])
            py1 = plsc.load_gather(blk_v, [lane, sub + 1])
            px2 = plsc.load_gather(blk_v, [lane, sub + 2])
            py2 = plsc.load_gather(blk_v, [lane, sub + 3])
            area1 = (px2 - px1) * (py2 - py1)

            def _gt_step(g, m):
                gfull = jnp.full((_LANES,), g, jnp.int32)
                gx1 = plsc.load_gather(gtb_v, [gfull, col0])
                gy1 = plsc.load_gather(gtb_v, [gfull, col0 + 1])
                gx2 = plsc.load_gather(gtb_v, [gfull, col0 + 2])
                gy2 = plsc.load_gather(gtb_v, [gfull, col0 + 3])
                w = jnp.maximum(jnp.minimum(px2, gx2) - jnp.maximum(px1, gx1), 0.0)
                h = jnp.maximum(jnp.minimum(py2, gy2) - jnp.maximum(py1, gy1), 0.0)
                inter = w * h
                area2 = (gx2 - gx1) * (gy2 - gy1)
                union = jnp.maximum(area1 + area2 - inter, 1e-6)
                iou = inter / union
                lab = plsc.load_gather(gtl_v, [gfull])
                sv = plsc.load_gather(crows_v, [lane, lab])
                cls = 1.0 / (1.0 + jnp.exp(-sv))
                iou2 = iou * iou
                return jnp.maximum(m, iou2 * iou2 * cls)

            m = lax.fori_loop(0, _G, _gt_step, jnp.zeros((_LANES,), jnp.float32))

            bits = plsc.bitcast(m, jnp.int32)
            seed_bits = (bits.astype(jnp.float32) * 0.2).astype(jnp.int32)
            y = plsc.bitcast(seed_bits + _FIFTH_ROOT_MAGIC, jnp.float32)
            for _ in range(3):
                y2 = y * y
                y4 = y2 * y2
                y = 0.8 * y + 0.2 * m / y4
            qbuf[...] = jnp.where(m > 0.0, y, 0.0)
            pltpu.async_copy(qbuf, out_hbm.at[idx_v], sem).wait()


@jax.jit
def _run(pos_i32, pblk, cls_scores, gtb, gtl_i32):
    mesh = plsc.VectorSubcoreMesh(core_axis_name="c", subcore_axis_name="s")
    f = pl.kernel(
        _sc_body,
        out_type=jax.ShapeDtypeStruct((_NPAD,), jnp.float32),
        mesh=mesh,
        compiler_params=pltpu.CompilerParams(
            needs_layout_passes=False, use_tc_tiling_on_sc=False),
        scratch_types=[
            pltpu.VMEM((_CHUNK,), jnp.float32),
            pltpu.VMEM((_LANES,), jnp.int32),
            pltpu.VMEM((_LANES,), jnp.int32),
            pltpu.VMEM((_LANES, 16), jnp.float32),
            pltpu.VMEM((_LANES, _C), jnp.float32),
            pltpu.VMEM((_G, 4), jnp.float32),
            pltpu.VMEM((_G,), jnp.int32),
            pltpu.VMEM((_LANES,), jnp.float32),
            pltpu.SemaphoreType.DMA,
        ],
    )
    return f(pos_i32, pblk, cls_scores, gtb, gtl_i32)


def kernel(bboxes, cls_scores, bbox_preds, gt_bboxes, bbox_levels, positive_inds, gt_labels):
    del bboxes, bbox_levels
    pos_i32 = positive_inds.astype(jnp.int32)
    gtl_i32 = gt_labels.astype(jnp.int32)
    pblk = bbox_preds.astype(jnp.float32).reshape(_N // 4, 16)
    gtb = gt_bboxes[:, :4].astype(jnp.float32)
    cls = cls_scores.astype(jnp.float32)
    out = _run(pos_i32, pblk, cls, gtb, gtl_i32)
    return out[:_N]


# speedup vs baseline: 1.0488x; 1.0488x over previous
import jax
import jax.numpy as jnp
from jax import lax
from jax.experimental import pallas as pl
from jax.experimental.pallas import tpu as pltpu
from jax.experimental.pallas import tpu_sc as plsc

_N = 20000
_G = 100
_C = 80
_P = 512
_LANES = 16
_WORKERS = 32
_FIFTH_ROOT_MAGIC = 851980270


def _sc_body(pos_hbm, pblk_hbm, cls_hbm, gtb_hbm, gtl_hbm, out_hbm,
             idx_v, blkidx_v, blk_v, crows_v, gtb_v, gtl_v, qbuf,
             sem_a, sem_b):
    c = lax.axis_index("c")
    s = lax.axis_index("s")
    wid = s * 2 + c

    pltpu.sync_copy(pos_hbm.at[pl.ds(wid * _LANES, _LANES)], idx_v)
    idx = idx_v[...]
    blkidx_v[...] = idx // 4
    gather_a = pltpu.async_copy(pblk_hbm.at[blkidx_v], blk_v, sem_a)
    gather_b = pltpu.async_copy(cls_hbm.at[idx_v], crows_v, sem_b)

    pltpu.sync_copy(gtb_hbm, gtb_v)
    pltpu.sync_copy(gtl_hbm, gtl_v)
    gather_a.wait()
    gather_b.wait()

    lane = lax.iota(jnp.int32, _LANES)
    sub = (idx % 4) * 4
    px1 = plsc.load_gather(blk_v, [lane, sub])
    py1 = plsc.load_gather(blk_v, [lane, sub + 1])
    px2 = plsc.load_gather(blk_v, [lane, sub + 2])
    py2 = plsc.load_gather(blk_v, [lane, sub + 3])
    area1 = (px2 - px1) * (py2 - py1)

    def _gt_step(g, m):
        gfull = jnp.full((_LANES,), g, jnp.int32)
        zero = jnp.zeros((_LANES,), jnp.int32)
        gx1 = plsc.load_gather(gtb_v, [gfull, zero])
        gy1 = plsc.load_gather(gtb_v, [gfull, zero + 1])
        gx2 = plsc.load_gather(gtb_v, [gfull, zero + 2])
        gy2 = plsc.load_gather(gtb_v, [gfull, zero + 3])
        w = jnp.maximum(jnp.minimum(px2, gx2) - jnp.maximum(px1, gx1), 0.0)
        h = jnp.maximum(jnp.minimum(py2, gy2) - jnp.maximum(py1, gy1), 0.0)
        inter = w * h
        area2 = (gx2 - gx1) * (gy2 - gy1)
        union = jnp.maximum(area1 + area2 - inter, 1e-6)
        iou = inter / union
        lab = plsc.load_gather(gtl_v, [gfull])
        sv = plsc.load_gather(crows_v, [lane, lab])
        cls = 1.0 / (1.0 + jnp.exp(-sv))
        iou2 = iou * iou
        return jnp.maximum(m, iou2 * iou2 * cls)

    m = lax.fori_loop(0, _G, _gt_step, jnp.zeros((_LANES,), jnp.float32),
                      unroll=4)

    bits = plsc.bitcast(m, jnp.int32)
    seed_bits = (bits.astype(jnp.float32) * 0.2).astype(jnp.int32)
    y = plsc.bitcast(seed_bits + _FIFTH_ROOT_MAGIC, jnp.float32)
    for _ in range(3):
        y2 = y * y
        y4 = y2 * y2
        y = 0.8 * y + 0.2 * m / y4
    qbuf[...] = jnp.where(m > 0.0, y, 0.0)
    pltpu.async_copy(qbuf, out_hbm.at[idx_v], sem_a).wait()


@jax.jit
def _run(pos_i32, pblk, cls_scores, gtb, gtl_i32):
    mesh = plsc.VectorSubcoreMesh(core_axis_name="c", subcore_axis_name="s")
    f = pl.kernel(
        _sc_body,
        out_type=(),
        mesh=mesh,
        compiler_params=pltpu.CompilerParams(
            needs_layout_passes=False, use_tc_tiling_on_sc=False),
        scratch_types=[
            pltpu.VMEM((_LANES,), jnp.int32),
            pltpu.VMEM((_LANES,), jnp.int32),
            pltpu.VMEM((_LANES, 16), jnp.float32),
            pltpu.VMEM((_LANES, _C), jnp.float32),
            pltpu.VMEM((_G, 4), jnp.float32),
            pltpu.VMEM((_G,), jnp.int32),
            pltpu.VMEM((_LANES,), jnp.float32),
            pltpu.SemaphoreType.DMA,
            pltpu.SemaphoreType.DMA,
        ],
    )
    out_ref = jax.new_ref(jnp.zeros((_N,), jnp.float32))
    f(pos_i32, pblk, cls_scores, gtb, gtl_i32, out_ref)
    return out_ref[...]


def kernel(bboxes, cls_scores, bbox_preds, gt_bboxes, bbox_levels, positive_inds, gt_labels):
    del bboxes, bbox_levels
    pos_i32 = positive_inds.astype(jnp.int32)
    gtl_i32 = gt_labels.astype(jnp.int32)
    pblk = bbox_preds.astype(jnp.float32).reshape(_N // 4, 16)
    gtb = gt_bboxes[:, :4].astype(jnp.float32)
    cls = cls_scores.astype(jnp.float32)
    return _run(pos_i32, pblk, cls, gtb, gtl_i32)
